# Initial kernel scaffold; baseline (speedup 1.0000x reference)
import jax
import jax.numpy as jnp
from jax.experimental import pallas as pl


def kernel(node_features, edge_index, edge_type, in_W, in_b, out_W, out_b, rg_W, rg_b, ug_W, ug_b, tf_W, tf_b, o_W, o_b):
    raise NotImplementedError("write your pallas kernel here")


# speedup vs baseline: 13.1267x; 13.1267x over previous
import functools

import jax
import jax.numpy as jnp
from jax import lax
from jax.experimental import pallas as pl
from jax.experimental.pallas import tpu as pltpu
from jax.experimental.pallas import tpu_sc as plsc

B, N, E = 4, 10000, 160000
D, H, OUT, T = 128, 128, 64, 3
DA = 144
NC, NS = 2, 16
K = 128
EPT = -(-E // (NS * K)) * K
NCH = EPT // K
EP = NS * EPT
ACC_ROWS = NCH * K
DUMP = N
RP = ACC_ROWS // NS


def _sc_scatter(x_hbm, gidx_hbm, sidx_hbm, zsrc_hbm, out_hbm,
                gi_v, si_v, rows_v, acc, sem):
    c = lax.axis_index("c")
    s = lax.axis_index("s")

    nfull = ACC_ROWS // K
    for b in range(B):
        for jj in range(nfull // NS):
            pltpu.sync_copy(zsrc_hbm, acc.at[pl.ds((jj * NS + s) * K, K)])
        rem = nfull - (nfull // NS) * NS
        if rem:
            @pl.when(s < rem)
            def _():
                pltpu.sync_copy(
                    zsrc_hbm, acc.at[pl.ds(((nfull // NS) * NS + s) * K, K)])

        pltpu.sync_copy(gidx_hbm.at[c, b, s], gi_v)
        pltpu.sync_copy(sidx_hbm.at[c, b, s], si_v)
        plsc.subcore_barrier()

        def chunk(j, carry):
            pltpu.async_copy(x_hbm.at[gi_v.at[j]], rows_v, sem).wait()
            pltpu.sync_copy(rows_v, acc.at[si_v.at[j]], add=True)
            return carry

        lax.fori_loop(0, NCH, chunk, 0)
        plsc.subcore_barrier()

        pltpu.sync_copy(acc.at[pl.ds(s * RP, RP)],
                        out_hbm.at[c, b, pl.ds(s * RP, RP)])
        plsc.subcore_barrier()


def _dense_body(x_ref, fi_ref, fo_ref, inWa_ref, outWa_ref,
                rgA_ref, rgB_ref, rgb_ref, ugA_ref, ugB_ref, ugb_ref,
                tfA_ref, tfB_ref, tfR_ref, tfb_ref, oW_ref, ob_ref, out_ref):
    b = pl.program_id(0)
    j = pl.program_id(1)
    x = x_ref[0]
    yin = fi_ref[0, 0] + x
    yout = fo_ref[0, 0] + x
    acc = jnp.zeros((1, OUT), jnp.float32)
    for t in range(T):
        a_in = jnp.dot(yin, inWa_ref[t], preferred_element_type=jnp.float32)
        a_out = jnp.dot(yout, outWa_ref[t], preferred_element_type=jnp.float32)
        r = jax.nn.sigmoid(
            jnp.dot(a_in, rgA_ref[...], preferred_element_type=jnp.float32)
            + jnp.dot(a_out, rgB_ref[...], preferred_element_type=jnp.float32)
            + rgb_ref[...])
        z = jax.nn.sigmoid(
            jnp.dot(a_in, ugA_ref[...], preferred_element_type=jnp.float32)
            + jnp.dot(a_out, ugB_ref[...], preferred_element_type=jnp.float32)
            + ugb_ref[...])
        h_hat = jnp.tanh(
            jnp.dot(a_in, tfA_ref[...], preferred_element_type=jnp.float32)
            + jnp.dot(a_out, tfB_ref[...], preferred_element_type=jnp.float32)
            + jnp.dot(r, tfR_ref[...], preferred_element_type=jnp.float32)
            + tfb_ref[...])
        prop = 1.0 - z + z * h_hat
        o = jnp.tanh(jnp.dot(prop, oW_ref[...],
                             preferred_element_type=jnp.float32) + ob_ref[...])
        acc = acc + jnp.sum(o, axis=0, keepdims=True)

    @pl.when((b == 0) & (j == 0))
    def _():
        out_ref[...] = jnp.zeros_like(out_ref)

    rows = lax.broadcasted_iota(jnp.int32, (8, OUT), 0)
    out_ref[...] += jnp.where(rows == b, jnp.broadcast_to(acc, (8, OUT)), 0.0)


def kernel(node_features, edge_index, edge_type, in_W, in_b, out_W, out_b,
           rg_W, rg_b, ug_W, ug_b, tf_W, tf_b, o_W, o_b):
    del edge_type
    f32 = jnp.float32

    row = edge_index[:, 0, :].astype(jnp.int32)
    col = edge_index[:, 1, :].astype(jnp.int32)
    off = (jnp.arange(B, dtype=jnp.int32) * N)[:, None]
    padg = jnp.broadcast_to(off, (B, EP - E))

    def pack(a):
        return a.reshape(B, NS, NCH, K)

    gidx = jnp.stack([pack(jnp.concatenate([col + off, padg], axis=1)),
                      pack(jnp.concatenate([row + off, padg], axis=1))])
    pads = jnp.full((B, EP - E), DUMP, jnp.int32)
    sidx = jnp.stack([pack(jnp.concatenate([row, pads], axis=1)),
                      pack(jnp.concatenate([col, pads], axis=1))])

    xaug = jnp.concatenate(
        [node_features,
         jnp.ones((B, N, 1), f32),
         jnp.zeros((B, N, DA - D - 1), f32)], axis=2)
    zsrc = jnp.zeros((K, DA), f32)

    mesh = plsc.VectorSubcoreMesh(core_axis_name="c", subcore_axis_name="s")
    sc_call = functools.partial(
        pl.kernel, _sc_scatter, mesh=mesh,
        compiler_params=pltpu.CompilerParams(use_tc_tiling_on_sc=False),
        out_type=jax.ShapeDtypeStruct((NC, B, ACC_ROWS, DA), f32),
        scratch_types=[
            pltpu.VMEM((NCH, K), jnp.int32),
            pltpu.VMEM((NCH, K), jnp.int32),
            pltpu.VMEM((K, DA), f32),
            pltpu.VMEM_SHARED((ACC_ROWS, DA), f32),
            pltpu.SemaphoreType.DMA,
        ])()
    feat = sc_call(xaug.reshape(B * N, DA), gidx, sidx, zsrc)

    def aug_w(W, bvec):
        z = jnp.zeros((T, DA - D - 1, W.shape[-1]), f32)
        return jnp.concatenate([W, bvec[:, None, :], z], axis=1)

    inWa = aug_w(in_W, in_b)
    outWa = aug_w(out_W, out_b)

    NB = 2000
    grid = (B, N // NB)
    full = lambda s: pl.BlockSpec(s, lambda b, j: (0,) * len(s))
    out = pl.pallas_call(
        _dense_body,
        grid=grid,
        in_specs=[
            pl.BlockSpec((1, NB, DA), lambda b, j: (b, j, 0)),
            pl.BlockSpec((1, 1, NB, DA), lambda b, j: (0, b, j, 0)),
            pl.BlockSpec((1, 1, NB, DA), lambda b, j: (1, b, j, 0)),
            full((T, DA, H)), full((T, DA, H)),
            full((H, H)), full((H, H)), full((1, H)),
            full((H, H)), full((H, H)), full((1, H)),
            full((H, H)), full((H, H)), full((H, H)), full((1, H)),
            full((H, OUT)), full((1, OUT)),
        ],
        out_specs=pl.BlockSpec((8, OUT), lambda b, j: (0, 0)),
        out_shape=jax.ShapeDtypeStruct((8, OUT), f32),
    )(xaug, feat, feat, inWa, outWa,
      rg_W[:H], rg_W[H:], rg_b.reshape(1, H),
      ug_W[:H], ug_W[H:], ug_b.reshape(1, H),
      tf_W[:H], tf_W[H:2 * H], tf_W[2 * H:], tf_b.reshape(1, H),
      o_W, o_b.reshape(1, OUT))
    return out[:B]
